# Initial kernel scaffold; baseline (speedup 1.0000x reference)
import jax
import jax.numpy as jnp
from jax.experimental import pallas as pl


def kernel(loc_preds, loc_targets, conf_preds, conf_targets):
    raise NotImplementedError("write your pallas kernel here")


# speedup vs baseline: 17.6529x; 17.6529x over previous
import jax
import jax.numpy as jnp
from jax import lax
from jax.experimental import pallas as pl
from jax.experimental.pallas import tpu as pltpu

_NUM_CLASSES = 21
_B = 128
_D = 8732
_BBLK = 8
_NSTEPS = _B // _BBLK
_MAX_FINITE_BITS = 0x7F800000


def _body(x_ref, y_ref, lp_ref, lt_ref, o0_ref, o1_ref, o2_ref,
          cl_ref, npos_ref, acc_ref):
    i = pl.program_id(0)

    @pl.when(i == 0)
    def _init():
        acc_ref[0] = 0.0
        acc_ref[1] = 0.0

    x = x_ref[...]
    y = y_ref[...]
    m = jnp.max(x, axis=0)
    s = jnp.sum(jnp.exp(x - m[None]), axis=0)
    lse = jnp.log(s) + m
    ks = lax.broadcasted_iota(jnp.int32, (_NUM_CLASSES, _BBLK, _D), 0)
    g = jnp.sum(jnp.where(y[None] == ks, x, 0.0), axis=0)
    ce = lse - g

    pos = y > 0
    posf = pos.astype(jnp.float32)
    npos_row = jnp.sum(posf, axis=1, keepdims=True)
    npos_ref[pl.ds(i * _BBLK, _BBLK), :] = npos_row
    acc_ref[1] += jnp.sum(ce * posf)

    cl = jnp.where(pos, 0.0, ce)
    cl_ref[pl.ds(i * _BBLK, _BBLK), :] = cl

    z = jnp.abs(lp_ref[...] - lt_ref[...])
    sl1 = jnp.where(z < 1.0, 0.5 * z * z, z - 0.5)
    acc_ref[0] += jnp.sum(sl1 * posf[None])

    @pl.when(i == _NSTEPS - 1)
    def _finish():
        cl_all = cl_ref[...]
        bits = lax.bitcast_convert_type(cl_all, jnp.int32)
        npos = npos_ref[...]
        k = jnp.minimum(3.0 * npos, float(_D - 1))

        def step(_, carry):
            lo, hi = carry
            mid = lo + lax.shift_right_logical(hi - lo + 1, 1)
            cnt = jnp.sum((bits >= mid).astype(jnp.float32), axis=1,
                          keepdims=True)
            ok = cnt >= k
            return jnp.where(ok, mid, lo), jnp.where(ok, hi, mid - 1)

        lo0 = jnp.zeros((_B, 1), jnp.int32)
        hi0 = jnp.full((_B, 1), _MAX_FINITE_BITS, jnp.int32)
        v_bits, _ = lax.fori_loop(0, 31, step, (lo0, hi0))

        above = bits > v_bits
        n_above = jnp.sum(above.astype(jnp.float32), axis=1, keepdims=True)
        sum_above = jnp.sum(jnp.where(above, cl_all, 0.0), axis=1,
                            keepdims=True)
        t = lax.bitcast_convert_type(v_bits, jnp.float32)
        tie = k - n_above
        conf_rows = sum_above + jnp.where(tie > 0.0, tie * t, 0.0)

        num_matched = jnp.sum(npos)
        conf_loss = (acc_ref[1] + jnp.sum(conf_rows)) / num_matched
        loc_loss = acc_ref[0] / num_matched
        o0_ref[...] = jnp.full((1, 1), loc_loss + conf_loss, jnp.float32)
        o1_ref[...] = jnp.full((1, 1), conf_loss, jnp.float32)
        o2_ref[...] = jnp.full((1, 1), loc_loss, jnp.float32)


def kernel(loc_preds, loc_targets, conf_preds, conf_targets):
    xt = jnp.transpose(conf_preds, (2, 0, 1))
    lpt = jnp.transpose(loc_preds, (2, 0, 1))
    ltt = jnp.transpose(loc_targets, (2, 0, 1))

    out_shape = [jax.ShapeDtypeStruct((1, 1), jnp.float32)] * 3
    o0, o1, o2 = pl.pallas_call(
        _body,
        grid=(_NSTEPS,),
        in_specs=[
            pl.BlockSpec((_NUM_CLASSES, _BBLK, _D), lambda i: (0, i, 0)),
            pl.BlockSpec((_BBLK, _D), lambda i: (i, 0)),
            pl.BlockSpec((4, _BBLK, _D), lambda i: (0, i, 0)),
            pl.BlockSpec((4, _BBLK, _D), lambda i: (0, i, 0)),
        ],
        out_specs=[pl.BlockSpec((1, 1), lambda i: (0, 0))] * 3,
        out_shape=out_shape,
        scratch_shapes=[
            pltpu.VMEM((_B, _D), jnp.float32),
            pltpu.VMEM((_B, 1), jnp.float32),
            pltpu.SMEM((2,), jnp.float32),
        ],
    )(xt, conf_targets, lpt, ltt)
    return (o0[0, 0], o1[0, 0], o2[0, 0])
